# Initial kernel scaffold; baseline (speedup 1.0000x reference)
import jax
import jax.numpy as jnp
from jax.experimental import pallas as pl


def kernel(inputs, tables):
    raise NotImplementedError("write your pallas kernel here")


# speedup vs baseline: 1.5199x; 1.5199x over previous
import jax
import jax.numpy as jnp
from jax import lax
from jax.experimental import pallas as pl
from jax.experimental.pallas import tpu as pltpu
from jax.experimental.pallas import tpu_sc as plsc

B = 16384
N_CAT = 26
VOCAB = 1000
EMB = 77
N_COLS = 41
NC = 2
NS = 16
NW = NC * NS
ROWS = B // NW
WORDS = ROWS * N_COLS
NVEC = WORDS // 16
CHUNK = 128
NCHUNK = WORDS // CHUNK


def _body(inp_hbm, tbl_hbm, out_hbm, inp_v, out_v, idx_v, gath_v, sem):
    wid = lax.axis_index("s") * NC + lax.axis_index("c")
    base = wid * WORDS
    pltpu.sync_copy(inp_hbm.at[pl.ds(base, WORDS)], inp_v)
    iota = lax.iota(jnp.int32, 16)

    def idx_body(k, carry):
        col = lax.rem(k * 16 + iota, N_COLS)
        val = inp_v[pl.ds(k * 16, 16)]
        tidx = (col * VOCAB + val) * EMB
        idx_v[pl.ds(k * 16, 16)] = jnp.where(col < N_CAT, tidx, 0)
        return carry
    lax.fori_loop(0, NVEC, idx_body, None)

    copies = []
    for j in range(NCHUNK):
        copies.append(pltpu.async_copy(
            tbl_hbm.at[idx_v.at[pl.ds(j * CHUNK, CHUNK)]],
            gath_v.at[pl.ds(j * CHUNK, CHUNK)], sem))
    for c in copies:
        c.wait()

    def merge_body(k, carry):
        col = lax.rem(k * 16 + iota, N_COLS)
        val = inp_v[pl.ds(k * 16, 16)].astype(jnp.float32)
        g = gath_v[pl.ds(k * 16, 16)]
        out_v[pl.ds(k * 16, 16)] = jnp.where(col < N_CAT, g, val)
        return carry
    lax.fori_loop(0, NVEC, merge_body, None)

    pltpu.sync_copy(out_v, out_hbm.at[pl.ds(base, WORDS)])


def kernel(inputs, tables):
    mesh = plsc.VectorSubcoreMesh(core_axis_name="c", subcore_axis_name="s")
    k = pl.kernel(
        _body,
        mesh=mesh,
        out_type=jax.ShapeDtypeStruct((B * N_COLS,), jnp.float32),
        scratch_types=[
            pltpu.VMEM((WORDS,), jnp.int32),
            pltpu.VMEM((WORDS,), jnp.float32),
            pltpu.VMEM((WORDS,), jnp.int32),
            pltpu.VMEM((WORDS,), jnp.float32),
            pltpu.SemaphoreType.DMA,
        ],
    )
    out_flat = k(inputs.reshape(-1), tables.reshape(-1))
    return out_flat.reshape(B, N_COLS)


# speedup vs baseline: 1.5235x; 1.0024x over previous
import jax
import jax.numpy as jnp
from jax import lax
from jax.experimental import pallas as pl
from jax.experimental.pallas import tpu as pltpu
from jax.experimental.pallas import tpu_sc as plsc

B = 16384
N_CAT = 26
VOCAB = 1000
EMB = 77
N_COLS = 41
NC = 2
NS = 16
NW = NC * NS
ROWS = B // NW
WORDS = ROWS * N_COLS
NVEC = WORDS // 16
CHUNK = WORDS
NCHUNK = WORDS // CHUNK


def _body(inp_hbm, tbl_hbm, out_hbm, inp_v, out_v, idx_v, gath_v, sem):
    wid = lax.axis_index("s") * NC + lax.axis_index("c")
    base = wid * WORDS
    pltpu.sync_copy(inp_hbm.at[pl.ds(base, WORDS)], inp_v)
    iota = lax.iota(jnp.int32, 16)

    def idx_body(k, carry):
        col = lax.rem(k * 16 + iota, N_COLS)
        val = inp_v[pl.ds(k * 16, 16)]
        tidx = (col * VOCAB + val) * EMB
        idx_v[pl.ds(k * 16, 16)] = jnp.where(col < N_CAT, tidx, 0)
        return carry
    lax.fori_loop(0, NVEC, idx_body, None)

    copies = []
    for j in range(NCHUNK):
        copies.append(pltpu.async_copy(
            tbl_hbm.at[idx_v.at[pl.ds(j * CHUNK, CHUNK)]],
            gath_v.at[pl.ds(j * CHUNK, CHUNK)], sem))
    for c in copies:
        c.wait()

    def merge_body(k, carry):
        col = lax.rem(k * 16 + iota, N_COLS)
        val = inp_v[pl.ds(k * 16, 16)].astype(jnp.float32)
        g = gath_v[pl.ds(k * 16, 16)]
        out_v[pl.ds(k * 16, 16)] = jnp.where(col < N_CAT, g, val)
        return carry
    lax.fori_loop(0, NVEC, merge_body, None)

    pltpu.sync_copy(out_v, out_hbm.at[pl.ds(base, WORDS)])


def kernel(inputs, tables):
    mesh = plsc.VectorSubcoreMesh(core_axis_name="c", subcore_axis_name="s")
    k = pl.kernel(
        _body,
        mesh=mesh,
        out_type=jax.ShapeDtypeStruct((B * N_COLS,), jnp.float32),
        scratch_types=[
            pltpu.VMEM((WORDS,), jnp.int32),
            pltpu.VMEM((WORDS,), jnp.float32),
            pltpu.VMEM((WORDS,), jnp.int32),
            pltpu.VMEM((WORDS,), jnp.float32),
            pltpu.SemaphoreType.DMA,
        ],
    )
    out_flat = k(inputs.reshape(-1), tables.reshape(-1))
    return out_flat.reshape(B, N_COLS)


# speedup vs baseline: 10.3436x; 6.7893x over previous
import jax
import jax.numpy as jnp
from jax import lax
from jax.experimental import pallas as pl
from jax.experimental.pallas import tpu as pltpu
from jax.experimental.pallas import tpu_sc as plsc

B = 16384
N_CAT = 26
VOCAB = 1000
EMB = 77
N_COLS = 41
NC = 2
NS = 16
NW = NC * NS
ROWS = B // NW
WORDS = ROWS * N_COLS
NVEC = WORDS // 16
CTAB = N_CAT * VOCAB
CT_PER = 1664
CT_VEC = CT_PER // 16


def _body(inp_hbm, tbl_hbm, out_hbm, inp_v, out_v, idx_v, gath_v, ctab_s, sem):
    sid = lax.axis_index("s")
    wid = sid * NC + lax.axis_index("c")
    base = wid * WORDS
    pltpu.sync_copy(inp_hbm.at[pl.ds(base, WORDS)], inp_v)
    iota = lax.iota(jnp.int32, 16)

    def ct_idx(j, carry):
        e = jnp.minimum(sid * CT_PER + j * 16 + iota, CTAB - 1)
        idx_v[pl.ds(j * 16, 16)] = e * EMB
        return carry
    lax.fori_loop(0, CT_VEC, ct_idx, None)
    pltpu.async_copy(tbl_hbm.at[idx_v.at[pl.ds(0, CT_PER)]],
                     gath_v.at[pl.ds(0, CT_PER)], sem).wait()
    pltpu.sync_copy(gath_v.at[pl.ds(0, CT_PER)],
                    ctab_s.at[pl.ds(sid * CT_PER, CT_PER)])

    def idx_body(k, carry):
        col = lax.rem(k * 16 + iota, N_COLS)
        val = inp_v[pl.ds(k * 16, 16)]
        tidx = col * VOCAB + val
        idx_v[pl.ds(k * 16, 16)] = jnp.where(col < N_CAT, tidx, 0)
        return carry
    lax.fori_loop(0, NVEC, idx_body, None)

    plsc.subcore_barrier()
    pltpu.async_copy(ctab_s.at[idx_v], gath_v, sem).wait()

    def merge_body(k, carry):
        col = lax.rem(k * 16 + iota, N_COLS)
        val = inp_v[pl.ds(k * 16, 16)].astype(jnp.float32)
        g = gath_v[pl.ds(k * 16, 16)]
        out_v[pl.ds(k * 16, 16)] = jnp.where(col < N_CAT, g, val)
        return carry
    lax.fori_loop(0, NVEC, merge_body, None)

    pltpu.sync_copy(out_v, out_hbm.at[pl.ds(base, WORDS)])


def kernel(inputs, tables):
    mesh = plsc.VectorSubcoreMesh(core_axis_name="c", subcore_axis_name="s")
    k = pl.kernel(
        _body,
        mesh=mesh,
        out_type=jax.ShapeDtypeStruct((B * N_COLS,), jnp.float32),
        scratch_types=[
            pltpu.VMEM((WORDS,), jnp.int32),
            pltpu.VMEM((WORDS,), jnp.float32),
            pltpu.VMEM((WORDS,), jnp.int32),
            pltpu.VMEM((WORDS,), jnp.float32),
            pltpu.VMEM_SHARED((NS * CT_PER,), jnp.float32),
            pltpu.SemaphoreType.DMA,
        ],
    )
    out_flat = k(inputs.reshape(-1), tables.reshape(-1))
    return out_flat.reshape(B, N_COLS)


# speedup vs baseline: 21.0432x; 2.0344x over previous
import jax
import jax.numpy as jnp
from jax import lax
from jax.experimental import pallas as pl
from jax.experimental.pallas import tpu as pltpu
from jax.experimental.pallas import tpu_sc as plsc

B = 16384
N_CAT = 26
VOCAB = 1000
EMB = 77
N_COLS = 41
NC = 2
NS = 16
NW = NC * NS
CATW = N_CAT * B // NW
NUMW = (N_COLS - N_CAT) * B // NW
NUM_BASE = N_CAT * B
CTAB = N_CAT * VOCAB
CT_PER = 1664
CT_VEC = CT_PER // 16
UNROLL = 4


def _body(inp_hbm, tbl_hbm, out_hbm, inp_cat_v, idx_v, gout_v,
          inp_num_v, out_num_v, ctidx_v, ctg_v, ctab_s, sem, sem2, sem3):
    sid = lax.axis_index("s")
    wid = sid * NC + lax.axis_index("c")
    cbase = wid * CATW
    nbase = NUM_BASE + wid * NUMW
    iota = lax.iota(jnp.int32, 16)

    a_cat = pltpu.async_copy(inp_hbm.at[pl.ds(cbase, CATW)], inp_cat_v, sem2)
    a_num = pltpu.async_copy(inp_hbm.at[pl.ds(nbase, NUMW)], inp_num_v, sem3)

    def ct_idx(j, carry):
        e = jnp.minimum(sid * CT_PER + j * 16 + iota, CTAB - 1)
        ctidx_v[pl.ds(j * 16, 16)] = e * EMB
        return carry
    lax.fori_loop(0, CT_VEC, ct_idx, None)
    a_ctab = pltpu.async_copy(tbl_hbm.at[ctidx_v], ctg_v, sem)

    a_cat.wait()

    def idx_body(k, carry):
        for u in range(UNROLL):
            off = k * (16 * UNROLL) + u * 16
            colv = ((cbase + off) >> 14) * VOCAB
            idx_v[pl.ds(off, 16)] = inp_cat_v[pl.ds(off, 16)] + colv
        return carry
    lax.fori_loop(0, CATW // (16 * UNROLL), idx_body, None)

    a_ctab.wait()
    pltpu.sync_copy(ctg_v, ctab_s.at[pl.ds(sid * CT_PER, CT_PER)])
    plsc.subcore_barrier()

    a_g = pltpu.async_copy(ctab_s.at[idx_v], gout_v, sem)

    def cast_body(k, carry):
        for u in range(UNROLL):
            off = k * (16 * UNROLL) + u * 16
            out_num_v[pl.ds(off, 16)] = (
                inp_num_v[pl.ds(off, 16)].astype(jnp.float32))
        return carry
    a_num.wait()
    lax.fori_loop(0, NUMW // (16 * UNROLL), cast_body, None)

    a_g.wait()
    pltpu.sync_copy(gout_v, out_hbm.at[pl.ds(cbase, CATW)])
    pltpu.sync_copy(out_num_v, out_hbm.at[pl.ds(nbase, NUMW)])


def kernel(inputs, tables):
    mesh = plsc.VectorSubcoreMesh(core_axis_name="c", subcore_axis_name="s")
    k = pl.kernel(
        _body,
        mesh=mesh,
        out_type=jax.ShapeDtypeStruct((N_COLS * B,), jnp.float32),
        scratch_types=[
            pltpu.VMEM((CATW,), jnp.int32),
            pltpu.VMEM((CATW,), jnp.int32),
            pltpu.VMEM((CATW,), jnp.float32),
            pltpu.VMEM((NUMW,), jnp.int32),
            pltpu.VMEM((NUMW,), jnp.float32),
            pltpu.VMEM((CT_PER,), jnp.int32),
            pltpu.VMEM((CT_PER,), jnp.float32),
            pltpu.VMEM_SHARED((NS * CT_PER,), jnp.float32),
            pltpu.SemaphoreType.DMA,
            pltpu.SemaphoreType.DMA,
            pltpu.SemaphoreType.DMA,
        ],
    )
    out_t = k(inputs.T.reshape(-1), tables.reshape(-1))
    return out_t.reshape(N_COLS, B).T
